# Initial kernel scaffold; baseline (speedup 1.0000x reference)
import jax
import jax.numpy as jnp
from jax.experimental import pallas as pl


def kernel(x, edge_index, edge_type, edge_norm, w_basis1, w_comp1, w_loop1, b1, w_basis2, w_comp2, w_loop2, b2):
    raise NotImplementedError("write your pallas kernel here")


# speedup vs baseline: 3.0249x; 3.0249x over previous
import functools

import jax
import jax.numpy as jnp
from jax import lax
from jax.experimental import pallas as pl
from jax.experimental.pallas import tpu as pltpu
from jax.experimental.pallas import tpu_sc as plsc

N = 10000
E = 320000
D = 128
R = 16
NB = 4

NC = 2
NS = 16
NW = NC * NS
CE = 80
EPW = E // NW
NCH = EPW // CE
SG = 25
NG = NCH // SG
GE = SG * CE
RPT = 632
RPT_LAST = N - 15 * RPT


def _wmix_body(comp_ref, basis_ref, w_ref):
    for r in range(R):
        acc = comp_ref[r, 0] * basis_ref[0]
        for b in range(1, NB):
            acc = acc + comp_ref[r, b] * basis_ref[b]
        w_ref[r] = acc


def _wmix(comp, basis):
    return pl.pallas_call(
        _wmix_body,
        in_specs=[
            pl.BlockSpec(memory_space=pltpu.SMEM),
            pl.BlockSpec(memory_space=pltpu.VMEM),
        ],
        out_specs=pl.BlockSpec(memory_space=pltpu.VMEM),
        out_shape=jax.ShapeDtypeStruct((R, D, D), jnp.float32),
    )(comp, basis)


BN = 2000


def _hrel_body(h_ref, w_ref, out_ref):
    out_ref[0] = jnp.dot(h_ref[...], w_ref[0], preferred_element_type=jnp.float32)


def _hrel(h, w):
    return pl.pallas_call(
        _hrel_body,
        grid=(N // BN, R),
        in_specs=[
            pl.BlockSpec((BN, D), lambda i, r: (i, 0)),
            pl.BlockSpec((1, D, D), lambda i, r: (r, 0, 0)),
        ],
        out_specs=pl.BlockSpec((1, BN, D), lambda i, r: (r, i, 0)),
        out_shape=jax.ShapeDtypeStruct((R, N, D), jnp.float32),
    )(h, w)


def _combine_body(agg_ref, h_ref, wl_ref, b_ref, out_ref, *, relu):
    y = agg_ref[0] + agg_ref[1]
    y = y + jnp.dot(h_ref[...], wl_ref[...], preferred_element_type=jnp.float32)
    y = y + b_ref[...]
    if relu:
        y = jnp.maximum(y, 0.0)
    out_ref[...] = y


def _combine(aggpair, h, w_loop, bias, relu):
    return pl.pallas_call(
        functools.partial(_combine_body, relu=relu),
        grid=(N // BN,),
        in_specs=[
            pl.BlockSpec((2, BN, D), lambda i: (0, i, 0)),
            pl.BlockSpec((BN, D), lambda i: (i, 0)),
            pl.BlockSpec((D, D), lambda i: (0, 0)),
            pl.BlockSpec((1, D), lambda i: (0, 0)),
        ],
        out_specs=pl.BlockSpec((BN, D), lambda i: (i, 0)),
        out_shape=jax.ShapeDtypeStruct((N, D), jnp.float32),
    )(aggpair, h, w_loop, bias)


def _sc_agg_body(ed_hbm, nrm_hbm, hrel_hbm, zeros_hbm, out_hbm,
                 ebuf, nbuf, sidx, dbuf, rows, agg_sh, sem):
    c = lax.axis_index("c")
    s = lax.axis_index("s")
    w = c * NS + s

    @pl.when(s < NS - 1)
    def _():
        pltpu.sync_copy(zeros_hbm.at[pl.ds(s * RPT, RPT)],
                        agg_sh.at[pl.ds(s * RPT, RPT)])

    @pl.when(s == NS - 1)
    def _():
        pltpu.sync_copy(zeros_hbm.at[pl.ds((NS - 1) * RPT, RPT_LAST)],
                        agg_sh.at[pl.ds((NS - 1) * RPT, RPT_LAST)])

    plsc.subcore_barrier()

    def group(kg, carry0):
        g = w * NG + kg
        pltpu.sync_copy(ed_hbm.at[g], ebuf)
        pltpu.sync_copy(nrm_hbm.at[g], nbuf)

        def chunk(k, carry):
            base = k * CE
            for t in range(CE // 16):
                sl = pl.ds(base + t * 16, 16)
                slo = pl.ds(t * 16, 16)
                sidx[slo] = ebuf[1, sl] * N + ebuf[0, sl]
                dbuf[slo] = ebuf[2, sl]
            pltpu.async_copy(hrel_hbm.at[sidx], rows, sem).wait()

            def edge16(t, c2):
                nvec = nbuf[0, pl.ds(base + t * 16, 16)]
                for l in range(16):
                    nv = nvec[l]
                    i = t * 16 + l
                    for j in range(D // 16):
                        sl2 = pl.ds(j * 16, 16)
                        rows[i, sl2] = rows[i, sl2] * nv
                return c2
            lax.fori_loop(0, CE // 16, edge16, 0)

            pltpu.sync_copy(rows, agg_sh.at[dbuf], add=True)
            return carry

        lax.fori_loop(0, SG, chunk, 0)
        return carry0

    lax.fori_loop(0, NG, group, 0)
    plsc.subcore_barrier()

    @pl.when(s < NS - 1)
    def _():
        pltpu.sync_copy(agg_sh.at[pl.ds(s * RPT, RPT)],
                        out_hbm.at[pl.ds(c * N + s * RPT, RPT)])

    @pl.when(s == NS - 1)
    def _():
        pltpu.sync_copy(agg_sh.at[pl.ds((NS - 1) * RPT, RPT_LAST)],
                        out_hbm.at[pl.ds(c * N + (NS - 1) * RPT, RPT_LAST)])


def _sc_agg(ed, nrm, hrel_flat, zeros):
    mesh = plsc.VectorSubcoreMesh(core_axis_name="c", subcore_axis_name="s")
    f = functools.partial(
        pl.kernel,
        mesh=mesh,
        out_type=jax.ShapeDtypeStruct((NC * N, D), jnp.float32),
        scratch_types=[
            pltpu.VMEM((3, GE), jnp.int32),
            pltpu.VMEM((1, GE), jnp.float32),
            pltpu.VMEM((CE,), jnp.int32),
            pltpu.VMEM((CE,), jnp.int32),
            pltpu.VMEM((CE, D), jnp.float32),
            pltpu.VMEM_SHARED((N, D), jnp.float32),
            pltpu.SemaphoreType.DMA,
        ],
    )(_sc_agg_body)
    return f(ed, nrm, hrel_flat, zeros)


def _layer(h, ed, nrm, zeros, w_basis, w_comp, w_loop, bias, relu):
    w = _wmix(w_comp, w_basis)
    hrel = _hrel(h, w).reshape(R * N, D)
    aggpair = _sc_agg(ed, nrm, hrel, zeros).reshape(NC, N, D)
    return _combine(aggpair, h, w_loop, bias.reshape(1, D), relu)


@jax.jit
def kernel(x, edge_index, edge_type, edge_norm,
           w_basis1, w_comp1, w_loop1, b1,
           w_basis2, w_comp2, w_loop2, b2):
    src = edge_index[0].astype(jnp.int32)
    dst = edge_index[1].astype(jnp.int32)
    ety = edge_type.astype(jnp.int32)
    ed = jnp.stack([src, ety, dst], axis=0)
    ed = ed.reshape(3, NW * NG, GE).transpose(1, 0, 2)
    nrm = edge_norm.astype(jnp.float32).reshape(NW * NG, 1, GE)
    zeros = jnp.zeros((N, D), jnp.float32)

    h1 = _layer(x, ed, nrm, zeros, w_basis1, w_comp1, w_loop1, b1, True)
    out = _layer(h1, ed, nrm, zeros, w_basis2, w_comp2, w_loop2, b2, False)
    return out
